# Initial kernel scaffold; baseline (speedup 1.0000x reference)
import functools

import jax
import jax.numpy as jnp
import numpy as np
from jax import lax
from jax.experimental import pallas as pl
from jax.experimental.pallas import tpu as pltpu
from jax.experimental.pallas import tpu_sc as plsc

COORD_DIM = 3
N_LEVELS = 16
F = 2
LOG2_T = 19
T = 1 << LOG2_T
N_MIN = 16
N_MAX = 512
BSZ = 131072
FACTOR = np.exp((np.log(N_MAX) - np.log(N_MIN)) / (N_LEVELS - 1))
P1 = np.int32(np.uint32(2654435761).astype(np.int64) - (1 << 32))
P2 = np.int32(805459861)

_RES = [float(np.floor(N_MIN * FACTOR**i)) for i in range(N_LEVELS)]
_LEVCONST = np.stack(
    [
        np.array([np.float32(2.0 / r) for r in _RES], dtype=np.float32),
        np.array([np.float32(r - 1.0) for r in _RES], dtype=np.float32),
    ]
)

NC = 2
NS = 16
NW = NC * NS
PW = BSZ // NW
PC = 2048
NCHUNK = PW // PC
NG = PC // 16
IDX_ROWS = PC * 8 // 128

_IOTA = np.arange(16, dtype=np.int32)
_LANEC = [np.arange(c * 16, c * 16 + 16, dtype=np.int32) for c in range(8)]
_Z16 = np.zeros(16, dtype=np.int32)
_O16 = np.ones(16, dtype=np.int32)


def _body(xT_hbm, tab_hbm, levc_hbm, out_hbm, x_v, levc_v, idx_v, rows_v, w_v,
          out_v, sem):
    cid = lax.axis_index("c")
    sid = lax.axis_index("s")
    wid = sid * NC + cid
    wbase = wid * PW

    for d in range(COORD_DIM):
        pltpu.sync_copy(xT_hbm.at[d, pl.ds(wbase, PW)], x_v.at[d])
    pltpu.sync_copy(levc_hbm, levc_v)

    iota = jnp.asarray(_IOTA)

    def level_body(l, carry):
        box = levc_v[0, l]
        resm1 = levc_v[1, l]
        box_s = jnp.full((16,), box, jnp.float32)
        resm1_s = jnp.full((16,), resm1, jnp.float32)
        lT = l * T

        def chunk_body(cc, carry2):
            cbase = cc * PC

            def ga(g, c3):
                o = cbase + g * 16
                hs = []
                ws = []
                for d in range(COORD_DIM):
                    xd = x_v[d, pl.ds(o, 16)]
                    t = (xd + 1.0) / box_s
                    tc = jnp.minimum(jnp.maximum(t, 0.0), resm1_s)
                    bl = tc.astype(jnp.int32)
                    w = t - bl.astype(jnp.float32)
                    ws.append(w)
                    if d == 0:
                        hs.append((bl, bl + 1))
                    elif d == 1:
                        m = bl * P1
                        hs.append((m, m + P1))
                    else:
                        m = bl * P2
                        hs.append((m, m + P2))
                for d in range(COORD_DIM):
                    w_v[d, g, :] = ws[d]
                for c in range(8):
                    h = hs[0][c & 1] ^ hs[1][(c >> 1) & 1] ^ hs[2][(c >> 2) & 1]
                    h = (h & (T - 1)) + lT
                    idx_v[g, pl.ds(c * 16, 16)] = h
                return c3

            lax.fori_loop(0, NG, ga, 0)

            pltpu.async_copy(tab_hbm.at[idx_v], rows_v, sem).wait()

            def gb(g, c3):
                w0 = w_v[0, g, :]
                w1 = w_v[1, g, :]
                w2 = w_v[2, g, :]
                u0 = 1.0 - w0
                u1 = 1.0 - w1
                u2 = 1.0 - w2
                sel = [(u0, w0), (u1, w1), (u2, w2)]
                g_s = jnp.full((16,), g, jnp.int32)
                z = jnp.asarray(_Z16)
                one = jnp.asarray(_O16)
                acc0 = None
                acc1 = None
                for c in range(8):
                    cw = (sel[0][c & 1] * sel[1][(c >> 1) & 1]) * sel[2][(c >> 2) & 1]
                    lane = jnp.asarray(_LANEC[c])
                    e0 = plsc.load_gather(rows_v, [g_s, lane, z])
                    e1 = plsc.load_gather(rows_v, [g_s, lane, one])
                    if acc0 is None:
                        acc0 = cw * e0
                        acc1 = cw * e1
                    else:
                        acc0 = acc0 + cw * e0
                        acc1 = acc1 + cw * e1
                row_ids = g_s * 16 + iota
                plsc.store_scatter(out_v, [row_ids, z], acc0)
                plsc.store_scatter(out_v, [row_ids, one], acc1)
                return c3

            lax.fori_loop(0, NG, gb, 0)

            pltpu.sync_copy(out_v, out_hbm.at[l, pl.ds(wbase + cbase, PC), :])
            return carry2

        lax.fori_loop(0, NCHUNK, chunk_body, 0)
        return carry

    lax.fori_loop(0, N_LEVELS, level_body, 0)


@jax.jit
def kernel(x, tables):
    xT = x.T
    tab2 = tables.reshape(N_LEVELS * T, F)
    levc = jnp.asarray(_LEVCONST)

    mesh = plsc.VectorSubcoreMesh(
        core_axis_name="c", subcore_axis_name="s", num_cores=NC, num_subcores=NS
    )
    call = pl.kernel(
        _body,
        out_type=jax.ShapeDtypeStruct((N_LEVELS, BSZ, F), jnp.float32),
        mesh=mesh,
        scratch_types=[
            pltpu.VMEM((COORD_DIM, PW), jnp.float32),
            pltpu.VMEM((2, 16), jnp.float32),
            pltpu.VMEM((IDX_ROWS, 128), jnp.int32),
            pltpu.VMEM((IDX_ROWS, 128, F), jnp.float32),
            pltpu.VMEM((COORD_DIM, NG, 16), jnp.float32),
            pltpu.VMEM((PC, F), jnp.float32),
            pltpu.SemaphoreType.DMA,
        ],
    )
    out = call(xT, tab2, levc)
    return out.transpose(1, 0, 2).reshape(BSZ, N_LEVELS * F)


# speedup vs baseline: 11.2168x; 11.2168x over previous
import jax
import jax.numpy as jnp
import numpy as np
from jax import lax
from jax.experimental import pallas as pl
from jax.experimental.pallas import tpu as pltpu
from jax.experimental.pallas import tpu_sc as plsc

COORD_DIM = 3
N_LEVELS = 16
F = 2
LOG2_T = 19
T = 1 << LOG2_T
N_MIN = 16
N_MAX = 512
BSZ = 131072
FACTOR = np.exp((np.log(N_MAX) - np.log(N_MIN)) / (N_LEVELS - 1))
P1 = np.int32(np.uint32(2654435761).astype(np.int64) - (1 << 32))
P2 = np.int32(805459861)

_RES = [float(np.floor(N_MIN * FACTOR**i)) for i in range(N_LEVELS)]
_LEVCONST = np.stack(
    [
        np.array([np.float32(2.0 / r) for r in _RES], dtype=np.float32),
        np.array([np.float32(r - 1.0) for r in _RES], dtype=np.float32),
    ]
)

NC = 2
NS = 16
NW = NC * NS
PW = BSZ // NW
PC = 1024
NCHUNK = PW // PC
NG = PC // 16
NIDX = PC * 8 * F


def _body(xT_hbm, tab_hbm, levc_hbm, out_hbm, x_v, levc_v, idx_v, rows_v, w_v,
          out_v, sem):
    cid = lax.axis_index("c")
    sid = lax.axis_index("s")
    wid = sid * NC + cid
    wbase = wid * PW

    for d in range(COORD_DIM):
        pltpu.sync_copy(
            xT_hbm.at[pl.ds(d * BSZ + wbase, PW)], x_v.at[pl.ds(d * PW, PW)]
        )
    pltpu.sync_copy(levc_hbm, levc_v)

    iota = lax.iota(jnp.int32, 16)

    def level_body(l, carry):
        l_s = jnp.full((16,), l, jnp.int32)
        box_s = plsc.load_gather(levc_v, [l_s])
        resm1_s = plsc.load_gather(levc_v, [l_s + 16])
        lT2 = l * (T * F)

        def chunk_body(cc, carry2):
            cbase = cc * PC

            def ga(g, c3):
                o = cbase + g * 16
                hs = []
                ws = []
                for d in range(COORD_DIM):
                    xd = x_v[pl.ds(d * PW + o, 16)]
                    t = (xd + 1.0) / box_s
                    tc = jnp.minimum(jnp.maximum(t, 0.0), resm1_s)
                    bl = tc.astype(jnp.int32)
                    w = t - bl.astype(jnp.float32)
                    ws.append(w)
                    if d == 0:
                        hs.append((bl, bl + 1))
                    elif d == 1:
                        m = bl * P1
                        hs.append((m, m + P1))
                    else:
                        m = bl * P2
                        hs.append((m, m + P2))
                for d in range(COORD_DIM):
                    w_v[pl.ds((d * NG + g) * 16, 16)] = ws[d]
                for c in range(8):
                    h = hs[0][c & 1] ^ hs[1][(c >> 1) & 1] ^ hs[2][(c >> 2) & 1]
                    h2 = ((h & (T - 1)) << 1) + lT2
                    idx_v[pl.ds(g * 256 + c * 32, 16)] = h2
                    idx_v[pl.ds(g * 256 + c * 32 + 16, 16)] = h2 + 1
                return c3

            lax.fori_loop(0, NG, ga, 0)

            pltpu.async_copy(tab_hbm.at[idx_v], rows_v, sem).wait()

            def gb(g, c3):
                w0 = w_v[pl.ds(g * 16, 16)]
                w1 = w_v[pl.ds((NG + g) * 16, 16)]
                w2 = w_v[pl.ds((2 * NG + g) * 16, 16)]
                u0 = 1.0 - w0
                u1 = 1.0 - w1
                u2 = 1.0 - w2
                sel = [(u0, w0), (u1, w1), (u2, w2)]
                o2 = g * 256
                acc0 = None
                acc1 = None
                for c in range(8):
                    cw = (sel[0][c & 1] * sel[1][(c >> 1) & 1]) * sel[2][(c >> 2) & 1]
                    e0 = rows_v[pl.ds(o2 + c * 32, 16)]
                    e1 = rows_v[pl.ds(o2 + c * 32 + 16, 16)]
                    if acc0 is None:
                        acc0 = cw * e0
                        acc1 = cw * e1
                    else:
                        acc0 = acc0 + cw * e0
                        acc1 = acc1 + cw * e1
                out_v[pl.ds(g * 16, 16)] = acc0
                out_v[pl.ds(PC + g * 16, 16)] = acc1
                return c3

            lax.fori_loop(0, NG, gb, 0)

            for f in range(F):
                pltpu.sync_copy(
                    out_v.at[pl.ds(f * PC, PC)],
                    out_hbm.at[pl.ds((l * F + f) * BSZ + wbase + cbase, PC)],
                )
            return carry2

        lax.fori_loop(0, NCHUNK, chunk_body, 0)
        return carry

    lax.fori_loop(0, N_LEVELS, level_body, 0)


@jax.jit
def kernel(x, tables):
    xT = x.T.reshape(-1)
    tabf = tables.reshape(-1)
    levc = jnp.asarray(_LEVCONST.reshape(-1))

    mesh = plsc.VectorSubcoreMesh(
        core_axis_name="c", subcore_axis_name="s", num_cores=NC, num_subcores=NS
    )
    call = pl.kernel(
        _body,
        out_type=jax.ShapeDtypeStruct((N_LEVELS * F * BSZ,), jnp.float32),
        mesh=mesh,
        compiler_params=pltpu.CompilerParams(
            needs_layout_passes=False, use_tc_tiling_on_sc=False
        ),
        scratch_types=[
            pltpu.VMEM((COORD_DIM * PW,), jnp.float32),
            pltpu.VMEM((2 * 16,), jnp.float32),
            pltpu.VMEM((NIDX,), jnp.int32),
            pltpu.VMEM((NIDX,), jnp.float32),
            pltpu.VMEM((COORD_DIM * NG * 16,), jnp.float32),
            pltpu.VMEM((F * PC,), jnp.float32),
            pltpu.SemaphoreType.DMA,
        ],
    )
    out = call(xT, tabf, levc)
    out = out.reshape(N_LEVELS * F, BSZ)
    return out.T.reshape(BSZ, N_LEVELS * F)


# speedup vs baseline: 11.3765x; 1.0142x over previous
import jax
import jax.numpy as jnp
import numpy as np
from jax import lax
from jax.experimental import pallas as pl
from jax.experimental.pallas import tpu as pltpu
from jax.experimental.pallas import tpu_sc as plsc

COORD_DIM = 3
N_LEVELS = 16
F = 2
LOG2_T = 19
T = 1 << LOG2_T
N_MIN = 16
N_MAX = 512
BSZ = 131072
FACTOR = np.exp((np.log(N_MAX) - np.log(N_MIN)) / (N_LEVELS - 1))
P1 = np.int32(np.uint32(2654435761).astype(np.int64) - (1 << 32))
P2 = np.int32(805459861)

_RES = [float(np.floor(N_MIN * FACTOR**i)) for i in range(N_LEVELS)]
_BOX = [np.float32(2.0 / r) for r in _RES]
_RESM1 = [np.float32(r - 1.0) for r in _RES]

NC = 2
NS = 16
NW = NC * NS
PW = BSZ // NW
PC = 1024
NCHUNK = PW // PC
NG = PC // 16
NIDX = PC * 8 * F
OD = N_LEVELS * F


def _body(x_hbm, tab_hbm, out_hbm, x_v, idx_v, rows_v, w_v, out_v, sem0, sem1):
    cid = lax.axis_index("c")
    sid = lax.axis_index("s")
    wid = sid * NC + cid
    wbase = wid * PW

    pltpu.sync_copy(x_hbm.at[pl.ds(wbase * COORD_DIM, PW * COORD_DIM)], x_v)

    iota = lax.iota(jnp.int32, 16)
    sems = (sem0, sem1)

    def chunk_body(cc, carry):
        cbase = cc * PC

        def phase_a(l):
            par = l % 2
            box_s = jnp.full((16,), _BOX[l], jnp.float32)
            resm1_s = jnp.full((16,), _RESM1[l], jnp.float32)
            lT2 = l * (T * F)

            def ga(g, c3):
                o = cbase + g * 16
                pos3 = (o + iota) * COORD_DIM
                hs = []
                ws = []
                for d in range(COORD_DIM):
                    xd = plsc.load_gather(x_v, [pos3 + d])
                    t = (xd + 1.0) / box_s
                    tc = jnp.minimum(jnp.maximum(t, 0.0), resm1_s)
                    bl = tc.astype(jnp.int32)
                    w = t - bl.astype(jnp.float32)
                    ws.append(w)
                    if d == 0:
                        hs.append((bl, bl + 1))
                    elif d == 1:
                        m = bl * P1
                        hs.append((m, m + P1))
                    else:
                        m = bl * P2
                        hs.append((m, m + P2))
                for d in range(COORD_DIM):
                    w_v[pl.ds(((par * COORD_DIM + d) * NG + g) * 16, 16)] = ws[d]
                for c in range(8):
                    h = hs[0][c & 1] ^ hs[1][(c >> 1) & 1] ^ hs[2][(c >> 2) & 1]
                    h2 = ((h & (T - 1)) << 1) + lT2
                    base = par * NIDX + g * 256 + c * 32
                    idx_v[pl.ds(base, 16)] = h2
                    idx_v[pl.ds(base + 16, 16)] = h2 + 1
                return c3

            lax.fori_loop(0, NG, ga, 0)
            return pltpu.async_copy(
                tab_hbm.at[idx_v.at[pl.ds(par * NIDX, NIDX)]],
                rows_v.at[pl.ds(par * NIDX, NIDX)],
                sems[par],
            )

        def phase_b(l):
            par = l % 2

            def gb(g, c3):
                w0 = w_v[pl.ds(((par * COORD_DIM + 0) * NG + g) * 16, 16)]
                w1 = w_v[pl.ds(((par * COORD_DIM + 1) * NG + g) * 16, 16)]
                w2 = w_v[pl.ds(((par * COORD_DIM + 2) * NG + g) * 16, 16)]
                u0 = 1.0 - w0
                u1 = 1.0 - w1
                u2 = 1.0 - w2
                sel = [(u0, w0), (u1, w1), (u2, w2)]
                o2 = par * NIDX + g * 256
                acc0 = None
                acc1 = None
                for c in range(8):
                    cw = (sel[0][c & 1] * sel[1][(c >> 1) & 1]) * sel[2][(c >> 2) & 1]
                    e0 = rows_v[pl.ds(o2 + c * 32, 16)]
                    e1 = rows_v[pl.ds(o2 + c * 32 + 16, 16)]
                    if acc0 is None:
                        acc0 = cw * e0
                        acc1 = cw * e1
                    else:
                        acc0 = acc0 + cw * e0
                        acc1 = acc1 + cw * e1
                pos = (g * 16 + iota) * OD + (2 * l)
                plsc.store_scatter(out_v, [pos], acc0)
                plsc.store_scatter(out_v, [pos + 1], acc1)
                return c3

            lax.fori_loop(0, NG, gb, 0)

        copies = {}
        copies[0] = phase_a(0)
        for l in range(1, N_LEVELS):
            copies[l] = phase_a(l)
            copies[l - 1].wait()
            phase_b(l - 1)
        copies[N_LEVELS - 1].wait()
        phase_b(N_LEVELS - 1)

        pltpu.sync_copy(out_v, out_hbm.at[pl.ds((wbase + cbase) * OD, PC * OD)])
        return carry

    lax.fori_loop(0, NCHUNK, chunk_body, 0)


@jax.jit
def kernel(x, tables):
    xf = x.reshape(-1)
    tabf = tables.reshape(-1)

    mesh = plsc.VectorSubcoreMesh(
        core_axis_name="c", subcore_axis_name="s", num_cores=NC, num_subcores=NS
    )
    call = pl.kernel(
        _body,
        out_type=jax.ShapeDtypeStruct((BSZ * OD,), jnp.float32),
        mesh=mesh,
        compiler_params=pltpu.CompilerParams(
            needs_layout_passes=False, use_tc_tiling_on_sc=False
        ),
        scratch_types=[
            pltpu.VMEM((COORD_DIM * PW,), jnp.float32),
            pltpu.VMEM((2 * NIDX,), jnp.int32),
            pltpu.VMEM((2 * NIDX,), jnp.float32),
            pltpu.VMEM((2 * COORD_DIM * NG * 16,), jnp.float32),
            pltpu.VMEM((PC * OD,), jnp.float32),
            pltpu.SemaphoreType.DMA,
            pltpu.SemaphoreType.DMA,
        ],
    )
    out = call(xf, tabf)
    return out.reshape(BSZ, OD)


# speedup vs baseline: 99.6434x; 8.7587x over previous
import jax
import jax.numpy as jnp
import numpy as np
from jax import lax
from jax.experimental import pallas as pl
from jax.experimental.pallas import tpu as pltpu
from jax.experimental.pallas import tpu_sc as plsc

COORD_DIM = 3
N_LEVELS = 16
F = 2
LOG2_T = 19
T = 1 << LOG2_T
N_MIN = 16
N_MAX = 512
BSZ = 131072
FACTOR = np.exp((np.log(N_MAX) - np.log(N_MIN)) / (N_LEVELS - 1))
P1 = np.int32(np.uint32(2654435761).astype(np.int64) - (1 << 32))
P2 = np.int32(805459861)

_RES = [float(np.floor(N_MIN * FACTOR**i)) for i in range(N_LEVELS)]
_BOX = [np.float32(2.0 / r) for r in _RES]
_RESM1 = [np.float32(r - 1.0) for r in _RES]

NC = 2
NS = 16
NW = NC * NS
PW = BSZ // NW
PC = 1024
NCHUNK = PW // PC
NG = PC // 16
NIDX = PC * 8 * F
OD = N_LEVELS * F


def _body(x_hbm, tab_hbm, out_hbm, x_v, idx_v, rows_v, w_v, out_v, sem0, sem1):
    cid = lax.axis_index("c")
    sid = lax.axis_index("s")
    wid = sid * NC + cid
    wbase = wid * PW

    pltpu.sync_copy(x_hbm.at[pl.ds(wbase * COORD_DIM, PW * COORD_DIM)], x_v)

    iota = lax.iota(jnp.int32, 16)
    sems = (sem0, sem1)

    def chunk_body(cc, carry):
        cbase = cc * PC

        def phase_a(l):
            par = l % 2
            box_s = jnp.full((16,), _BOX[l], jnp.float32)
            resm1_s = jnp.full((16,), _RESM1[l], jnp.float32)
            lbase = l * (T * F)

            def ga(g, c3):
                o = cbase + g * 16
                pos3 = (o + iota) * COORD_DIM
                hs = []
                ws = []
                for d in range(COORD_DIM):
                    xd = plsc.load_gather(x_v, [pos3 + d])
                    t = (xd + 1.0) / box_s
                    tc = jnp.minimum(jnp.maximum(t, 0.0), resm1_s)
                    bl = tc.astype(jnp.int32)
                    w = t - bl.astype(jnp.float32)
                    ws.append(w)
                    if d == 0:
                        hs.append((bl, bl + 1))
                    elif d == 1:
                        m = bl * P1
                        hs.append((m, m + P1))
                    else:
                        m = bl * P2
                        hs.append((m, m + P2))
                for d in range(COORD_DIM):
                    w_v[pl.ds(((par * COORD_DIM + d) * NG + g) * 16, 16)] = ws[d]
                for c in range(8):
                    h = hs[0][c & 1] ^ hs[1][(c >> 1) & 1] ^ hs[2][(c >> 2) & 1]
                    h = h & (T - 1)
                    a0 = (((h >> 7) << 8) + (h & 127)) + lbase
                    base = par * NIDX + g * 256 + c * 32
                    idx_v[pl.ds(base, 16)] = a0
                    idx_v[pl.ds(base + 16, 16)] = a0 + 128
                return c3

            lax.fori_loop(0, NG, ga, 0)
            return pltpu.async_copy(
                tab_hbm.at[idx_v.at[pl.ds(par * NIDX, NIDX)]],
                rows_v.at[pl.ds(par * NIDX, NIDX)],
                sems[par],
            )

        def phase_b(l):
            par = l % 2

            def gb(g, c3):
                w0 = w_v[pl.ds(((par * COORD_DIM + 0) * NG + g) * 16, 16)]
                w1 = w_v[pl.ds(((par * COORD_DIM + 1) * NG + g) * 16, 16)]
                w2 = w_v[pl.ds(((par * COORD_DIM + 2) * NG + g) * 16, 16)]
                u0 = 1.0 - w0
                u1 = 1.0 - w1
                u2 = 1.0 - w2
                sel = [(u0, w0), (u1, w1), (u2, w2)]
                o2 = par * NIDX + g * 256
                acc0 = None
                acc1 = None
                for c in range(8):
                    cw = (sel[0][c & 1] * sel[1][(c >> 1) & 1]) * sel[2][(c >> 2) & 1]
                    e0 = rows_v[pl.ds(o2 + c * 32, 16)]
                    e1 = rows_v[pl.ds(o2 + c * 32 + 16, 16)]
                    if acc0 is None:
                        acc0 = cw * e0
                        acc1 = cw * e1
                    else:
                        acc0 = acc0 + cw * e0
                        acc1 = acc1 + cw * e1
                ob = (g >> 3) * 1024 + (g & 7) * 16
                c0 = 2 * l
                off0 = (c0 >> 3) * 8192 + (c0 & 7) * 128
                off1 = ((c0 + 1) >> 3) * 8192 + ((c0 + 1) & 7) * 128
                out_v[pl.ds(off0 + ob, 16)] = acc0
                out_v[pl.ds(off1 + ob, 16)] = acc1
                return c3

            lax.fori_loop(0, NG, gb, 0)

        copies = {}
        copies[0] = phase_a(0)
        for l in range(1, N_LEVELS):
            copies[l] = phase_a(l)
            copies[l - 1].wait()
            phase_b(l - 1)
        copies[N_LEVELS - 1].wait()
        phase_b(N_LEVELS - 1)

        for cb in range(4):
            pltpu.sync_copy(
                out_v.at[pl.ds(cb * 8192, 8192)],
                out_hbm.at[pl.ds(cb * 1048576 + (wbase + cbase) * 8, 8192)],
            )
        return carry

    lax.fori_loop(0, NCHUNK, chunk_body, 0)


@jax.jit
def kernel(x, tables):
    xf = x.reshape(-1)
    tabf = tables.reshape(N_LEVELS, T // 128, 128, F).transpose(0, 1, 3, 2).reshape(-1)

    mesh = plsc.VectorSubcoreMesh(
        core_axis_name="c", subcore_axis_name="s", num_cores=NC, num_subcores=NS
    )
    call = pl.kernel(
        _body,
        out_type=jax.ShapeDtypeStruct((BSZ * OD,), jnp.float32),
        mesh=mesh,
        compiler_params=pltpu.CompilerParams(
            needs_layout_passes=False, use_tc_tiling_on_sc=False
        ),
        scratch_types=[
            pltpu.VMEM((COORD_DIM * PW,), jnp.float32),
            pltpu.VMEM((2 * NIDX,), jnp.int32),
            pltpu.VMEM((2 * NIDX,), jnp.float32),
            pltpu.VMEM((2 * COORD_DIM * NG * 16,), jnp.float32),
            pltpu.VMEM((PC * OD,), jnp.float32),
            pltpu.SemaphoreType.DMA,
            pltpu.SemaphoreType.DMA,
        ],
    )
    out = call(xf, tabf)
    out = out.reshape(OD // 8, BSZ // 128, 8, 128)
    return out.transpose(1, 3, 0, 2).reshape(BSZ, OD)


# speedup vs baseline: 256.9481x; 2.5787x over previous
import jax
import jax.numpy as jnp
import numpy as np
from jax import lax
from jax.experimental import pallas as pl
from jax.experimental.pallas import tpu as pltpu
from jax.experimental.pallas import tpu_sc as plsc

COORD_DIM = 3
N_LEVELS = 16
F = 2
LOG2_T = 19
T = 1 << LOG2_T
N_MIN = 16
N_MAX = 512
BSZ = 131072
FACTOR = np.exp((np.log(N_MAX) - np.log(N_MIN)) / (N_LEVELS - 1))
P1 = np.int32(np.uint32(2654435761).astype(np.int64) - (1 << 32))
P2 = np.int32(805459861)

_RES = [float(np.floor(N_MIN * FACTOR**i)) for i in range(N_LEVELS)]
_BOX = [np.float32(2.0 / r) for r in _RES]
_RESM1 = [np.float32(r - 1.0) for r in _RES]

NC = 2
NS = 16
NW = NC * NS
PW = BSZ // NW
PC = 512
NCHUNK = PW // PC
NG = PC // 16
NIDX = PC * 8 * F
OD = N_LEVELS * F
SLAB = T * F


def _body(x_hbm, tab_hbm, out_hbm, x_v, idx_v, rows_v, w_v, out_v, spm, gsem,
          ssem):
    cid = lax.axis_index("c")
    sid = lax.axis_index("s")
    wid = sid * NC + cid
    wbase = wid * PW

    pltpu.sync_copy(x_hbm.at[pl.ds(wbase * COORD_DIM, PW * COORD_DIM)], x_v)

    iota = lax.iota(jnp.int32, 16)

    def make_phase_a(l):
        box_s = jnp.full((16,), _BOX[l], jnp.float32)
        resm1_s = jnp.full((16,), _RESM1[l], jnp.float32)

        def phase_a(cc):
            par = cc & 1
            cbase = cc * PC

            def ga(g, c3):
                o = cbase + g * 16
                pos3 = (o + iota) * COORD_DIM
                hs = []
                ws = []
                for d in range(COORD_DIM):
                    xd = plsc.load_gather(x_v, [pos3 + d])
                    t = (xd + 1.0) / box_s
                    tc = jnp.minimum(jnp.maximum(t, 0.0), resm1_s)
                    bl = tc.astype(jnp.int32)
                    w = t - bl.astype(jnp.float32)
                    ws.append(w)
                    if d == 0:
                        hs.append((bl, bl + 1))
                    elif d == 1:
                        m = bl * P1
                        hs.append((m, m + P1))
                    else:
                        m = bl * P2
                        hs.append((m, m + P2))
                for d in range(COORD_DIM):
                    w_v[pl.ds((par * COORD_DIM + d) * (NG * 16) + g * 16, 16)] = ws[d]
                for c in range(8):
                    h = hs[0][c & 1] ^ hs[1][(c >> 1) & 1] ^ hs[2][(c >> 2) & 1]
                    h = h & (T - 1)
                    a0 = ((h >> 7) << 8) + (h & 127)
                    base = par * NIDX + g * 256 + c * 32
                    idx_v[pl.ds(base, 16)] = a0
                    idx_v[pl.ds(base + 16, 16)] = a0 + 128
                return c3

            lax.fori_loop(0, NG, ga, 0)
            pltpu.async_copy(
                spm.at[idx_v.at[pl.ds(par * NIDX, NIDX)]],
                rows_v.at[pl.ds(par * NIDX, NIDX)],
                gsem,
            )

        return phase_a

    def gather_wait(cc):
        par = cc & 1
        pltpu.make_async_copy(
            spm.at[idx_v.at[pl.ds(par * NIDX, NIDX)]],
            rows_v.at[pl.ds(par * NIDX, NIDX)],
            gsem,
        ).wait()

    def make_phase_b(l):
        cbq = l >> 2
        ci0 = (2 * l) & 7

        def phase_b(cc):
            par = cc & 1
            cbase = cc * PC

            def gb(g, c3):
                w0 = w_v[pl.ds((par * COORD_DIM + 0) * (NG * 16) + g * 16, 16)]
                w1 = w_v[pl.ds((par * COORD_DIM + 1) * (NG * 16) + g * 16, 16)]
                w2 = w_v[pl.ds((par * COORD_DIM + 2) * (NG * 16) + g * 16, 16)]
                u0 = 1.0 - w0
                u1 = 1.0 - w1
                u2 = 1.0 - w2
                sel = [(u0, w0), (u1, w1), (u2, w2)]
                o2 = par * NIDX + g * 256
                acc0 = None
                acc1 = None
                for c in range(8):
                    cw = (sel[0][c & 1] * sel[1][(c >> 1) & 1]) * sel[2][(c >> 2) & 1]
                    e0 = rows_v[pl.ds(o2 + c * 32, 16)]
                    e1 = rows_v[pl.ds(o2 + c * 32 + 16, 16)]
                    if acc0 is None:
                        acc0 = cw * e0
                        acc1 = cw * e1
                    else:
                        acc0 = acc0 + cw * e0
                        acc1 = acc1 + cw * e1
                ob = (g >> 3) * 256 + (g & 7) * 16
                out_v[pl.ds(ob, 16)] = acc0
                out_v[pl.ds(ob + 128, 16)] = acc1
                return c3

            lax.fori_loop(0, NG, gb, 0)

            for pb in range(PC // 128):
                pltpu.sync_copy(
                    out_v.at[pl.ds(pb * 256, 256)],
                    out_hbm.at[
                        pl.ds(
                            cbq * (BSZ * 8)
                            + (wbase + cbase + pb * 128) * 8
                            + ci0 * 128,
                            256,
                        )
                    ],
                )

        return phase_b

    for l in range(N_LEVELS):
        plsc.subcore_barrier()

        @pl.when(sid == 0)
        def _(l=l):
            pltpu.async_copy(
                tab_hbm.at[pl.ds(l * SLAB, SLAB)], spm, ssem
            ).wait()

        plsc.subcore_barrier()

        phase_a = make_phase_a(l)
        phase_b = make_phase_b(l)

        phase_a(0)

        def chunk_body(cc, carry, phase_a=phase_a, phase_b=phase_b):
            phase_a(cc)
            gather_wait(cc - 1)
            phase_b(cc - 1)
            return carry

        lax.fori_loop(1, NCHUNK, chunk_body, 0)
        gather_wait(NCHUNK - 1)
        phase_b(NCHUNK - 1)


@jax.jit
def kernel(x, tables):
    xf = x.reshape(-1)
    tabf = tables.reshape(N_LEVELS, T // 128, 128, F).transpose(0, 1, 3, 2).reshape(-1)

    mesh = plsc.VectorSubcoreMesh(
        core_axis_name="c", subcore_axis_name="s", num_cores=NC, num_subcores=NS
    )
    call = pl.kernel(
        _body,
        out_type=jax.ShapeDtypeStruct((BSZ * OD,), jnp.float32),
        mesh=mesh,
        compiler_params=pltpu.CompilerParams(
            needs_layout_passes=False, use_tc_tiling_on_sc=False
        ),
        scratch_types=[
            pltpu.VMEM((COORD_DIM * PW,), jnp.float32),
            pltpu.VMEM((2 * NIDX,), jnp.int32),
            pltpu.VMEM((2 * NIDX,), jnp.float32),
            pltpu.VMEM((2 * COORD_DIM * NG * 16,), jnp.float32),
            pltpu.VMEM((F * PC,), jnp.float32),
            pltpu.VMEM_SHARED((SLAB,), jnp.float32),
            pltpu.SemaphoreType.DMA,
            pltpu.SemaphoreType.DMA,
        ],
    )
    out = call(xf, tabf)
    out = out.reshape(OD // 8, BSZ // 128, 8, 128)
    return out.transpose(1, 3, 0, 2).reshape(BSZ, OD)
